# baseline (speedup 1.0000x reference)
import functools

import jax
import jax.numpy as jnp
from jax import lax
from jax.experimental import pallas as pl
from jax.experimental.pallas import tpu as pltpu
from jax.experimental.pallas import tpu_sc as plsc

V = 1024
D = 64
D2 = 2 * D
BATCH = 16
SEQ = 4096
B = BATCH * SEQ
NC, NS = 2, 16
NW = NC * NS
IDS_PER_ROW = 128
ROWS_TOTAL = B // IDS_PER_ROW
STAGES = 4
ROWS_PER_STAGE = ROWS_TOTAL // STAGES
B_STAGE = B // STAGES
BATCH_STAGE = BATCH // STAGES
ROWS_PER_W = ROWS_PER_STAGE // NW
ROWS_PER_CHUNK = 2
IDS_PER_CHUNK = ROWS_PER_CHUNK * IDS_PER_ROW
CHUNKS = ROWS_PER_W // ROWS_PER_CHUNK
IDS_PER_W = ROWS_PER_W * IDS_PER_ROW
NBUF = 2
SPLIT_BLK = 2048


def _tables_body(inv_ref, tab_ref):
    f = inv_ref[...]
    tab_ref[:, :D] = jnp.cos(f).astype(jnp.bfloat16)
    tab_ref[:, D:] = jnp.sin(f).astype(jnp.bfloat16)


def _make_tables(inv_freq):
    return pl.pallas_call(
        _tables_body,
        out_shape=jax.ShapeDtypeStruct((V, D2), jnp.bfloat16),
    )(inv_freq)


def _gather_body(stage, tab, idx_hbm, comb_out, idx_v, buf, gsem0, gsem1,
                 wsem0, wsem1):
    wid = lax.axis_index("s") * NC + lax.axis_index("c")
    base = wid * IDS_PER_W
    row_base = stage * ROWS_PER_STAGE + wid * ROWS_PER_W
    gsems = (gsem0, gsem1)
    wsems = (wsem0, wsem1)

    def outer(g, carry):
        for b in range(NBUF):
            c = g * NBUF + b
            row0 = row_base + c * ROWS_PER_CHUNK
            off = base + c * IDS_PER_CHUNK
            dst = comb_out.at[pl.ds(off, IDS_PER_CHUNK)]

            @pl.when(g >= 1)
            def _():
                pltpu.make_async_copy(buf.at[b], dst, wsems[b]).wait()

            pltpu.sync_copy(idx_hbm.at[pl.ds(row0, ROWS_PER_CHUNK)],
                            idx_v.at[b])
            cps = []
            for j in range(ROWS_PER_CHUNK):
                d = pl.ds(j * IDS_PER_ROW, IDS_PER_ROW)
                cps.append(pltpu.async_copy(
                    tab.at[idx_v.at[b, j]], buf.at[b, d], gsems[b]))
            for cp in cps:
                cp.wait()
            pltpu.async_copy(buf.at[b], dst, wsems[b])
        return carry

    lax.fori_loop(0, CHUNKS // NBUF, outer, 0)

    for b in range(NBUF):
        c = CHUNKS - NBUF + b
        off = base + c * IDS_PER_CHUNK
        dst = comb_out.at[pl.ds(off, IDS_PER_CHUNK)]
        pltpu.make_async_copy(buf.at[b], dst, wsems[b]).wait()


@functools.cache
def _make_gather(stage):
    return pl.kernel(
        functools.partial(_gather_body, stage),
        out_type=jax.ShapeDtypeStruct((B_STAGE, D2), jnp.bfloat16),
        mesh=plsc.VectorSubcoreMesh(core_axis_name="c", subcore_axis_name="s"),
        compiler_params=pltpu.CompilerParams(use_tc_tiling_on_sc=False),
        scratch_types=[
            pltpu.VMEM((NBUF, ROWS_PER_CHUNK, IDS_PER_ROW), jnp.int32),
            pltpu.VMEM((NBUF, IDS_PER_CHUNK, D2), jnp.bfloat16),
            pltpu.SemaphoreType.DMA,
            pltpu.SemaphoreType.DMA,
            pltpu.SemaphoreType.DMA,
            pltpu.SemaphoreType.DMA,
        ],
    )


_BLKS_PER_BATCH = SEQ // SPLIT_BLK


def _split_first_body(comb_ref, cos_ref, sin_ref):
    rows = comb_ref[0].astype(jnp.float32)
    cos_ref[0] = rows[:, :D].T
    sin_ref[0] = rows[:, D:].T


def _split_rest_body(comb_ref, cos_in_ref, sin_in_ref, cos_ref, sin_ref):
    del cos_in_ref, sin_in_ref
    rows = comb_ref[0].astype(jnp.float32)
    cos_ref[0] = rows[:, :D].T
    sin_ref[0] = rows[:, D:].T


def _out_spec(stage):
    return pl.BlockSpec(
        (1, D, SPLIT_BLK),
        lambda i, _s=stage: (_s * BATCH_STAGE + i // _BLKS_PER_BATCH, 0,
                             i % _BLKS_PER_BATCH))


_COMB_SPEC = pl.BlockSpec(
    (1, SPLIT_BLK, D2),
    lambda i: (i // _BLKS_PER_BATCH, i % _BLKS_PER_BATCH, 0))

_OUT_SHAPE = (jax.ShapeDtypeStruct((BATCH, D, SEQ), jnp.float32),
              jax.ShapeDtypeStruct((BATCH, D, SEQ), jnp.float32))


def _split_stage(stage, comb, cos_t=None, sin_t=None):
    comb3 = comb.reshape(BATCH_STAGE, SEQ, D2)
    if stage == 0:
        return pl.pallas_call(
            _split_first_body,
            grid=(B_STAGE // SPLIT_BLK,),
            in_specs=[_COMB_SPEC],
            out_specs=(_out_spec(0),) * 2,
            out_shape=_OUT_SHAPE,
        )(comb3)
    return pl.pallas_call(
        _split_rest_body,
        grid=(B_STAGE // SPLIT_BLK,),
        in_specs=[
            _COMB_SPEC,
            pl.BlockSpec(memory_space=pl.ANY),
            pl.BlockSpec(memory_space=pl.ANY),
        ],
        out_specs=(_out_spec(stage),) * 2,
        out_shape=_OUT_SHAPE,
        input_output_aliases={1: 0, 2: 1},
    )(comb3, cos_t, sin_t)


def kernel(x, position_ids, inv_freq):
    tab = _make_tables(inv_freq.astype(jnp.float32))
    idx = position_ids.reshape(ROWS_TOTAL, IDS_PER_ROW).astype(jnp.int32)
    combs = [_make_gather(s)(tab, idx) for s in range(STAGES)]
    cos_t = sin_t = None
    for s in range(STAGES):
        cos_t, sin_t = _split_stage(s, combs[s], cos_t, sin_t)
    return (jnp.swapaxes(cos_t, 1, 2).astype(x.dtype),
            jnp.swapaxes(sin_t, 1, 2).astype(x.dtype))


# speedup vs baseline: 1.6466x; 1.6466x over previous
import functools

import jax
import jax.numpy as jnp
from jax import lax
from jax.experimental import pallas as pl
from jax.experimental.pallas import tpu as pltpu
from jax.experimental.pallas import tpu_sc as plsc

V = 1024
D = 64
D2 = 2 * D
BATCH = 16
SEQ = 4096
B = BATCH * SEQ
NC, NS = 2, 16
NW = NC * NS
IDS_PER_ROW = 128
ROWS_TOTAL = B // IDS_PER_ROW
SC_STAGES = 2
ROWS_PER_STAGE = 128
B_STAGE = ROWS_PER_STAGE * IDS_PER_ROW
BATCH_STAGE = 4
TC_BATCH0 = 8
MM_C = 512
ROWS_PER_W = ROWS_PER_STAGE // NW
ROWS_PER_CHUNK = 2
IDS_PER_CHUNK = ROWS_PER_CHUNK * IDS_PER_ROW
CHUNKS = ROWS_PER_W // ROWS_PER_CHUNK
IDS_PER_W = ROWS_PER_W * IDS_PER_ROW
NBUF = 2
SPLIT_BLK = 2048


def _tables_body(inv_ref, tab_ref, tab_t_ref):
    f = inv_ref[...]
    c = jnp.cos(f)
    s = jnp.sin(f)
    tab_ref[:, :D] = c
    tab_ref[:, D:] = s
    tab_t_ref[:D, :] = c.T.astype(jnp.bfloat16)
    tab_t_ref[D:, :] = s.T.astype(jnp.bfloat16)


def _make_tables(inv_freq):
    return pl.pallas_call(
        _tables_body,
        out_shape=(jax.ShapeDtypeStruct((V, D2), jnp.float32),
                   jax.ShapeDtypeStruct((D2, V), jnp.bfloat16)),
    )(inv_freq)


def _gather_body(stage, tab, idx_hbm, comb_out, idx_v, buf, gsem0, gsem1,
                 wsem0, wsem1):
    wid = lax.axis_index("s") * NC + lax.axis_index("c")
    base = wid * IDS_PER_W
    row_base = stage * ROWS_PER_STAGE + wid * ROWS_PER_W
    gsems = (gsem0, gsem1)
    wsems = (wsem0, wsem1)

    def outer(g, carry):
        for b in range(NBUF):
            c = g * NBUF + b
            row0 = row_base + c * ROWS_PER_CHUNK
            off = base + c * IDS_PER_CHUNK
            dst = comb_out.at[pl.ds(off, IDS_PER_CHUNK)]

            @pl.when(g >= 1)
            def _():
                pltpu.make_async_copy(buf.at[b], dst, wsems[b]).wait()

            pltpu.sync_copy(idx_hbm.at[pl.ds(row0, ROWS_PER_CHUNK)],
                            idx_v.at[b])
            cps = []
            for j in range(ROWS_PER_CHUNK):
                d = pl.ds(j * IDS_PER_ROW, IDS_PER_ROW)
                cps.append(pltpu.async_copy(
                    tab.at[idx_v.at[b, j]], buf.at[b, d], gsems[b]))
            for cp in cps:
                cp.wait()
            pltpu.async_copy(buf.at[b], dst, wsems[b])
        return carry

    lax.fori_loop(0, CHUNKS // NBUF, outer, 0)

    for b in range(NBUF):
        c = CHUNKS - NBUF + b
        off = base + c * IDS_PER_CHUNK
        dst = comb_out.at[pl.ds(off, IDS_PER_CHUNK)]
        pltpu.make_async_copy(buf.at[b], dst, wsems[b]).wait()


@functools.cache
def _make_gather(stage):
    return pl.kernel(
        functools.partial(_gather_body, stage),
        out_type=jax.ShapeDtypeStruct((B_STAGE, D2), jnp.float32),
        mesh=plsc.VectorSubcoreMesh(core_axis_name="c", subcore_axis_name="s"),
        compiler_params=pltpu.CompilerParams(use_tc_tiling_on_sc=False),
        scratch_types=[
            pltpu.VMEM((NBUF, ROWS_PER_CHUNK, IDS_PER_ROW), jnp.int32),
            pltpu.VMEM((NBUF, IDS_PER_CHUNK, D2), jnp.float32),
            pltpu.SemaphoreType.DMA,
            pltpu.SemaphoreType.DMA,
            pltpu.SemaphoreType.DMA,
            pltpu.SemaphoreType.DMA,
        ],
    )


_BLKS_PER_BATCH = SEQ // SPLIT_BLK


def _split_first_body(comb_ref, cos_ref, sin_ref):
    rows = comb_ref[0]
    cos_ref[0] = rows[:, :D].T
    sin_ref[0] = rows[:, D:].T


def _split_rest_body(comb_ref, cos_in_ref, sin_in_ref, cos_ref, sin_ref):
    del cos_in_ref, sin_in_ref
    rows = comb_ref[0]
    cos_ref[0] = rows[:, :D].T
    sin_ref[0] = rows[:, D:].T


def _out_spec(stage):
    return pl.BlockSpec(
        (1, D, SPLIT_BLK),
        lambda i, _s=stage: (_s * BATCH_STAGE + i // _BLKS_PER_BATCH, 0,
                             i % _BLKS_PER_BATCH))


_COMB_SPEC = pl.BlockSpec(
    (1, SPLIT_BLK, D2),
    lambda i: (i // _BLKS_PER_BATCH, i % _BLKS_PER_BATCH, 0))

_OUT_SHAPE = (jax.ShapeDtypeStruct((BATCH, D, SEQ), jnp.float32),
              jax.ShapeDtypeStruct((BATCH, D, SEQ), jnp.float32))


def _split_stage(stage, comb, cos_t, sin_t):
    comb3 = comb.reshape(BATCH_STAGE, SEQ, D2)
    return pl.pallas_call(
        _split_rest_body,
        grid=(B_STAGE // SPLIT_BLK,),
        in_specs=[
            _COMB_SPEC,
            pl.BlockSpec(memory_space=pl.ANY),
            pl.BlockSpec(memory_space=pl.ANY),
        ],
        out_specs=(_out_spec(stage),) * 2,
        out_shape=_OUT_SHAPE,
        input_output_aliases={1: 0, 2: 1},
    )(comb3, cos_t, sin_t)


def _matmul_body(tab_t_ref, idx_ref, cos_ref, sin_ref):
    idx = idx_ref[0, 0]
    onehot = (lax.broadcasted_iota(jnp.int32, (V, MM_C), 0)
              == idx[None, :]).astype(jnp.bfloat16)
    res = jnp.dot(tab_t_ref[...], onehot,
                  preferred_element_type=jnp.float32)
    cos_ref[0] = res[:D]
    sin_ref[0] = res[D:]


_MM_BLKS = SEQ // MM_C


def _matmul(tab_t, idx3):
    return pl.pallas_call(
        _matmul_body,
        grid=((BATCH - TC_BATCH0) * _MM_BLKS,),
        in_specs=[
            pl.BlockSpec((D2, V), lambda i: (0, 0)),
            pl.BlockSpec((1, 1, MM_C),
                         lambda i: (TC_BATCH0 * _MM_BLKS + i, 0, 0)),
        ],
        out_specs=(pl.BlockSpec(
            (1, D, MM_C),
            lambda i: (TC_BATCH0 + i // _MM_BLKS, 0, i % _MM_BLKS)),) * 2,
        out_shape=_OUT_SHAPE,
    )(tab_t, idx3)


def kernel(x, position_ids, inv_freq):
    tab, tab_t = _make_tables(inv_freq.astype(jnp.float32))
    idx = position_ids.reshape(ROWS_TOTAL, IDS_PER_ROW).astype(jnp.int32)
    idx3 = position_ids.reshape(BATCH * _MM_BLKS, 1, MM_C).astype(jnp.int32)
    combs = [_make_gather(s)(tab, idx) for s in range(SC_STAGES)]
    cos_t, sin_t = _matmul(tab_t, idx3)
    for s in range(SC_STAGES):
        cos_t, sin_t = _split_stage(s, combs[s], cos_t, sin_t)
    return (jnp.swapaxes(cos_t, 1, 2).astype(x.dtype),
            jnp.swapaxes(sin_t, 1, 2).astype(x.dtype))


# speedup vs baseline: 2.0727x; 1.2588x over previous
import functools

import jax
import jax.numpy as jnp
from jax import lax
from jax.experimental import pallas as pl
from jax.experimental.pallas import tpu as pltpu
from jax.experimental.pallas import tpu_sc as plsc

V = 1024
D = 64
D2 = 2 * D
BATCH = 16
SEQ = 4096
B = BATCH * SEQ
NC, NS = 2, 16
NW = NC * NS
IDS_PER_ROW = 128
ROWS_TOTAL = B // IDS_PER_ROW
SC_STAGES = 2
ROWS_PER_STAGE = 128
B_STAGE = ROWS_PER_STAGE * IDS_PER_ROW
BATCH_STAGE = 4
TC_BATCH0 = 8
MM_C = 2048
ROWS_PER_W = ROWS_PER_STAGE // NW
ROWS_PER_CHUNK = 2
IDS_PER_CHUNK = ROWS_PER_CHUNK * IDS_PER_ROW
CHUNKS = ROWS_PER_W // ROWS_PER_CHUNK
IDS_PER_W = ROWS_PER_W * IDS_PER_ROW
NBUF = 2
SPLIT_BLK = 2048


def _tables_body(inv_ref, tab_ref, tab_t_ref):
    f = inv_ref[...]
    c = jnp.cos(f)
    s = jnp.sin(f)
    tab_ref[:, :D] = c
    tab_ref[:, D:] = s
    tab_t_ref[:D, :] = c.T.astype(jnp.bfloat16)
    tab_t_ref[D:, :] = s.T.astype(jnp.bfloat16)


def _make_tables(inv_freq):
    return pl.pallas_call(
        _tables_body,
        out_shape=(jax.ShapeDtypeStruct((V, D2), jnp.float32),
                   jax.ShapeDtypeStruct((D2, V), jnp.bfloat16)),
    )(inv_freq)


def _gather_body(stage, tab, idx_hbm, comb_out, idx_v, buf, gsem0, gsem1,
                 wsem0, wsem1):
    wid = lax.axis_index("s") * NC + lax.axis_index("c")
    base = wid * IDS_PER_W
    row_base = stage * ROWS_PER_STAGE + wid * ROWS_PER_W
    gsems = (gsem0, gsem1)
    wsems = (wsem0, wsem1)

    def outer(g, carry):
        for b in range(NBUF):
            c = g * NBUF + b
            row0 = row_base + c * ROWS_PER_CHUNK
            off = base + c * IDS_PER_CHUNK
            dst = comb_out.at[pl.ds(off, IDS_PER_CHUNK)]

            @pl.when(g >= 1)
            def _():
                pltpu.make_async_copy(buf.at[b], dst, wsems[b]).wait()

            pltpu.sync_copy(idx_hbm.at[pl.ds(row0, ROWS_PER_CHUNK)],
                            idx_v.at[b])
            cps = []
            for j in range(ROWS_PER_CHUNK):
                d = pl.ds(j * IDS_PER_ROW, IDS_PER_ROW)
                cps.append(pltpu.async_copy(
                    tab.at[idx_v.at[b, j]], buf.at[b, d], gsems[b]))
            for cp in cps:
                cp.wait()
            pltpu.async_copy(buf.at[b], dst, wsems[b])
        return carry

    lax.fori_loop(0, CHUNKS // NBUF, outer, 0)

    for b in range(NBUF):
        c = CHUNKS - NBUF + b
        off = base + c * IDS_PER_CHUNK
        dst = comb_out.at[pl.ds(off, IDS_PER_CHUNK)]
        pltpu.make_async_copy(buf.at[b], dst, wsems[b]).wait()


@functools.cache
def _make_gather(stage):
    return pl.kernel(
        functools.partial(_gather_body, stage),
        out_type=jax.ShapeDtypeStruct((B_STAGE, D2), jnp.float32),
        mesh=plsc.VectorSubcoreMesh(core_axis_name="c", subcore_axis_name="s"),
        compiler_params=pltpu.CompilerParams(use_tc_tiling_on_sc=False),
        scratch_types=[
            pltpu.VMEM((NBUF, ROWS_PER_CHUNK, IDS_PER_ROW), jnp.int32),
            pltpu.VMEM((NBUF, IDS_PER_CHUNK, D2), jnp.float32),
            pltpu.SemaphoreType.DMA,
            pltpu.SemaphoreType.DMA,
            pltpu.SemaphoreType.DMA,
            pltpu.SemaphoreType.DMA,
        ],
    )


_BLKS_PER_BATCH = SEQ // SPLIT_BLK


def _split_first_body(comb_ref, cos_ref, sin_ref):
    rows = comb_ref[0]
    cos_ref[0] = rows[:, :D].T
    sin_ref[0] = rows[:, D:].T


def _split_rest_body(comb_ref, cos_in_ref, sin_in_ref, cos_ref, sin_ref):
    del cos_in_ref, sin_in_ref
    rows = comb_ref[0]
    cos_ref[0] = rows[:, :D].T
    sin_ref[0] = rows[:, D:].T


def _out_spec(stage):
    return pl.BlockSpec(
        (1, D, SPLIT_BLK),
        lambda i, _s=stage: (_s * BATCH_STAGE + i // _BLKS_PER_BATCH, 0,
                             i % _BLKS_PER_BATCH))


_COMB_SPEC = pl.BlockSpec(
    (1, SPLIT_BLK, D2),
    lambda i: (i // _BLKS_PER_BATCH, i % _BLKS_PER_BATCH, 0))

_OUT_SHAPE = (jax.ShapeDtypeStruct((BATCH, D, SEQ), jnp.float32),
              jax.ShapeDtypeStruct((BATCH, D, SEQ), jnp.float32))


def _split_stage(stage, comb, cos_t, sin_t):
    comb3 = comb.reshape(BATCH_STAGE, SEQ, D2)
    return pl.pallas_call(
        _split_rest_body,
        grid=(B_STAGE // SPLIT_BLK,),
        in_specs=[
            _COMB_SPEC,
            pl.BlockSpec(memory_space=pl.ANY),
            pl.BlockSpec(memory_space=pl.ANY),
        ],
        out_specs=(_out_spec(stage),) * 2,
        out_shape=_OUT_SHAPE,
        input_output_aliases={1: 0, 2: 1},
    )(comb3, cos_t, sin_t)


def _matmul_body(tab_t_ref, idx_ref, cos_ref, sin_ref):
    idx = idx_ref[0, 0]
    onehot = (lax.broadcasted_iota(jnp.int32, (V, MM_C), 0)
              == idx[None, :]).astype(jnp.bfloat16)
    res = jnp.dot(tab_t_ref[...], onehot,
                  preferred_element_type=jnp.float32)
    cos_ref[0] = res[:D]
    sin_ref[0] = res[D:]


_MM_BLKS = SEQ // MM_C


def _matmul(tab_t, idx3):
    return pl.pallas_call(
        _matmul_body,
        grid=((BATCH - TC_BATCH0) * _MM_BLKS,),
        in_specs=[
            pl.BlockSpec((D2, V), lambda i: (0, 0)),
            pl.BlockSpec((1, 1, MM_C),
                         lambda i: (TC_BATCH0 * _MM_BLKS + i, 0, 0)),
        ],
        out_specs=(pl.BlockSpec(
            (1, D, MM_C),
            lambda i: (TC_BATCH0 + i // _MM_BLKS, 0, i % _MM_BLKS)),) * 2,
        out_shape=_OUT_SHAPE,
    )(tab_t, idx3)


def kernel(x, position_ids, inv_freq):
    tab, tab_t = _make_tables(inv_freq.astype(jnp.float32))
    idx = position_ids.reshape(ROWS_TOTAL, IDS_PER_ROW).astype(jnp.int32)
    idx3 = position_ids.reshape(BATCH * _MM_BLKS, 1, MM_C).astype(jnp.int32)
    combs = [_make_gather(s)(tab, idx) for s in range(SC_STAGES)]
    cos_t, sin_t = _matmul(tab_t, idx3)
    for s in range(SC_STAGES):
        cos_t, sin_t = _split_stage(s, combs[s], cos_t, sin_t)
    return (jnp.swapaxes(cos_t, 1, 2).astype(x.dtype),
            jnp.swapaxes(sin_t, 1, 2).astype(x.dtype))


# speedup vs baseline: 2.2366x; 1.0791x over previous
import functools

import jax
import jax.numpy as jnp
from jax import lax
from jax.experimental import pallas as pl
from jax.experimental.pallas import tpu as pltpu
from jax.experimental.pallas import tpu_sc as plsc

V = 1024
D = 64
D2 = 2 * D
BATCH = 16
SEQ = 4096
B = BATCH * SEQ
NC, NS = 2, 16
NW = NC * NS
IDS_PER_ROW = 128
ROWS_TOTAL = B // IDS_PER_ROW
SC_STAGES = 2
ROWS_PER_STAGE = 128
B_STAGE = ROWS_PER_STAGE * IDS_PER_ROW
BATCH_STAGE = 4
TC_BATCH0 = 8
MM_C = 4096
ROWS_PER_W = ROWS_PER_STAGE // NW
ROWS_PER_CHUNK = 2
IDS_PER_CHUNK = ROWS_PER_CHUNK * IDS_PER_ROW
CHUNKS = ROWS_PER_W // ROWS_PER_CHUNK
IDS_PER_W = ROWS_PER_W * IDS_PER_ROW
NBUF = 2
SPLIT_BLK = 2048


def _tables_body(inv_t_ref, tab_ref, tab_t_ref):
    ft = inv_t_ref[...]
    ct = jnp.cos(ft)
    st = jnp.sin(ft)
    tab_ref[:, :D] = ct.T
    tab_ref[:, D:] = st.T
    tab_t_ref[:D, :] = ct.astype(jnp.bfloat16)
    tab_t_ref[D:, :] = st.astype(jnp.bfloat16)


def _make_tables(inv_freq):
    return pl.pallas_call(
        _tables_body,
        out_shape=(jax.ShapeDtypeStruct((V, D2), jnp.float32),
                   jax.ShapeDtypeStruct((D2, V), jnp.bfloat16)),
    )(jnp.swapaxes(inv_freq, 0, 1))


def _gather_body(stage, tab, idx_hbm, comb_out, idx_v, buf, gsem0, gsem1,
                 wsem0, wsem1):
    wid = lax.axis_index("s") * NC + lax.axis_index("c")
    base = wid * IDS_PER_W
    row_base = stage * ROWS_PER_STAGE + wid * ROWS_PER_W
    gsems = (gsem0, gsem1)
    wsems = (wsem0, wsem1)

    def outer(g, carry):
        for b in range(NBUF):
            c = g * NBUF + b
            row0 = row_base + c * ROWS_PER_CHUNK
            off = base + c * IDS_PER_CHUNK
            dst = comb_out.at[pl.ds(off, IDS_PER_CHUNK)]

            @pl.when(g >= 1)
            def _():
                pltpu.make_async_copy(buf.at[b], dst, wsems[b]).wait()

            pltpu.sync_copy(idx_hbm.at[pl.ds(row0, ROWS_PER_CHUNK)],
                            idx_v.at[b])
            cps = []
            for j in range(ROWS_PER_CHUNK):
                d = pl.ds(j * IDS_PER_ROW, IDS_PER_ROW)
                cps.append(pltpu.async_copy(
                    tab.at[idx_v.at[b, j]], buf.at[b, d], gsems[b]))
            for cp in cps:
                cp.wait()
            pltpu.async_copy(buf.at[b], dst, wsems[b])
        return carry

    lax.fori_loop(0, CHUNKS // NBUF, outer, 0)

    for b in range(NBUF):
        c = CHUNKS - NBUF + b
        off = base + c * IDS_PER_CHUNK
        dst = comb_out.at[pl.ds(off, IDS_PER_CHUNK)]
        pltpu.make_async_copy(buf.at[b], dst, wsems[b]).wait()


@functools.cache
def _make_gather(stage):
    return pl.kernel(
        functools.partial(_gather_body, stage),
        out_type=jax.ShapeDtypeStruct((B_STAGE, D2), jnp.float32),
        mesh=plsc.VectorSubcoreMesh(core_axis_name="c", subcore_axis_name="s"),
        compiler_params=pltpu.CompilerParams(use_tc_tiling_on_sc=False),
        scratch_types=[
            pltpu.VMEM((NBUF, ROWS_PER_CHUNK, IDS_PER_ROW), jnp.int32),
            pltpu.VMEM((NBUF, IDS_PER_CHUNK, D2), jnp.float32),
            pltpu.SemaphoreType.DMA,
            pltpu.SemaphoreType.DMA,
            pltpu.SemaphoreType.DMA,
            pltpu.SemaphoreType.DMA,
        ],
    )


_BLKS_PER_BATCH = SEQ // SPLIT_BLK


def _split_first_body(comb_ref, cos_ref, sin_ref):
    rows = comb_ref[0]
    cos_ref[0] = rows[:, :D].T
    sin_ref[0] = rows[:, D:].T


def _split_rest_body(comb_ref, cos_in_ref, sin_in_ref, cos_ref, sin_ref):
    del cos_in_ref, sin_in_ref
    rows = comb_ref[0]
    cos_ref[0] = rows[:, :D].T
    sin_ref[0] = rows[:, D:].T


def _out_spec(stage):
    return pl.BlockSpec(
        (1, D, SPLIT_BLK),
        lambda i, _s=stage: (_s * BATCH_STAGE + i // _BLKS_PER_BATCH, 0,
                             i % _BLKS_PER_BATCH))


_COMB_SPEC = pl.BlockSpec(
    (1, SPLIT_BLK, D2),
    lambda i: (i // _BLKS_PER_BATCH, i % _BLKS_PER_BATCH, 0))

_OUT_SHAPE = (jax.ShapeDtypeStruct((BATCH, D, SEQ), jnp.float32),
              jax.ShapeDtypeStruct((BATCH, D, SEQ), jnp.float32))


def _split_stage(stage, comb, cos_t, sin_t):
    comb3 = comb.reshape(BATCH_STAGE, SEQ, D2)
    return pl.pallas_call(
        _split_rest_body,
        grid=(B_STAGE // SPLIT_BLK,),
        in_specs=[
            _COMB_SPEC,
            pl.BlockSpec(memory_space=pl.ANY),
            pl.BlockSpec(memory_space=pl.ANY),
        ],
        out_specs=(_out_spec(stage),) * 2,
        out_shape=_OUT_SHAPE,
        input_output_aliases={1: 0, 2: 1},
    )(comb3, cos_t, sin_t)


def _matmul_body(tab_t_ref, idx_ref, cos_ref, sin_ref):
    idx = idx_ref[0, 0]
    onehot = (lax.broadcasted_iota(jnp.int32, (V, MM_C), 0)
              == idx[None, :]).astype(jnp.bfloat16)
    res = jnp.dot(tab_t_ref[...], onehot,
                  preferred_element_type=jnp.float32)
    cos_ref[0] = res[:D]
    sin_ref[0] = res[D:]


_MM_BLKS = SEQ // MM_C


def _matmul(tab_t, idx3):
    return pl.pallas_call(
        _matmul_body,
        grid=((BATCH - TC_BATCH0) * _MM_BLKS,),
        in_specs=[
            pl.BlockSpec((D2, V), lambda i: (0, 0)),
            pl.BlockSpec((1, 1, MM_C),
                         lambda i: (TC_BATCH0 * _MM_BLKS + i, 0, 0)),
        ],
        out_specs=(pl.BlockSpec(
            (1, D, MM_C),
            lambda i: (TC_BATCH0 + i // _MM_BLKS, 0, i % _MM_BLKS)),) * 2,
        out_shape=_OUT_SHAPE,
    )(tab_t, idx3)


def kernel(x, position_ids, inv_freq):
    tab, tab_t = _make_tables(inv_freq.astype(jnp.float32))
    idx = position_ids.reshape(ROWS_TOTAL, IDS_PER_ROW).astype(jnp.int32)
    idx3 = position_ids.reshape(BATCH * _MM_BLKS, 1, MM_C).astype(jnp.int32)
    combs = [_make_gather(s)(tab, idx) for s in range(SC_STAGES)]
    cos_t, sin_t = _matmul(tab_t, idx3)
    for s in range(SC_STAGES):
        cos_t, sin_t = _split_stage(s, combs[s], cos_t, sin_t)
    return (jnp.swapaxes(cos_t, 1, 2).astype(x.dtype),
            jnp.swapaxes(sin_t, 1, 2).astype(x.dtype))
